# baseline (speedup 1.0000x reference)
import jax
import jax.numpy as jnp
from jax import lax
from jax.experimental import pallas as pl
from jax.experimental.pallas import tpu as pltpu
from jax.experimental.pallas import tpu_sc as plsc

_B = 64
_L = 512
_G = 8
_D = 128
_CAT = 20
_CL = 16
_NCHUNK = _L // _CL
_NBC = _CL * _G
_NC, _NS = 2, 16
_NW = _NC * _NS
_BPW = _B // _NW
_NE = (8192 - 1) * 8192 + 1


def _sc_body(nb_hbm, we_hbm, x_hbm, emb_hbm, edge_hbm, elast_hbm, node_hbm,
             out_hbm,
             nbv, wev, xidx,
             idxn0, idxn1, idxn2, idxn3, idxe0, idxe1, idxe2, idxe3,
             flg0, flg1, flg2, flg3,
             rows0, rows1, rows2, rows3, nrows0, nrows1, nrows2, nrows3,
             wec0, wec1, wec2, wec3, wnc0, wnc1, wnc2, wnc3,
             elv, accb, sem0, sem1, sem2, sem3):
    bufs = [
        (idxn0, idxe0, flg0, rows0, nrows0, wec0, wnc0, sem0),
        (idxn1, idxe1, flg1, rows1, nrows1, wec1, wnc1, sem1),
        (idxn2, idxe2, flg2, rows2, nrows2, wec2, wnc2, sem2),
        (idxn3, idxe3, flg3, rows3, nrows3, wec3, wnc3, sem3),
    ]
    wid = lax.axis_index("s") * _NC + lax.axis_index("c")
    lane = lax.broadcasted_iota(jnp.int32, (16,), 0)
    zero16 = jnp.zeros((16,), jnp.int32)
    pltpu.sync_copy(elast_hbm, elv)
    lastv = plsc.load_gather(elv, [zero16, zero16])

    def issue(j, p):
        idxn, idxe, flg, rows, nrows, wec, wnc, sem = bufs[p]
        for g in range(_G):
            sc_idx = lane * _G + g
            plsc.store_scatter(idxn, [sc_idx], nbv[g, pl.ds(j * _CL, _CL)])
            ev = wev[g, pl.ds(j * _CL, _CL)]
            plsc.store_scatter(idxe, [sc_idx], jnp.minimum(ev, _NE - 2))
            plsc.store_scatter(
                flg, [sc_idx],
                jnp.where(ev == _NE - 1,
                          jnp.full((16,), 1.0, jnp.float32),
                          jnp.zeros((16,), jnp.float32)))
        xsl = xidx.at[pl.ds(j * _CL, _CL)]
        pltpu.async_copy(emb_hbm.at[idxn], rows, sem)
        pltpu.async_copy(emb_hbm.at[xsl], nrows, sem)
        pltpu.async_copy(edge_hbm.at[idxe], wec, sem)
        pltpu.async_copy(node_hbm.at[xsl], wnc, sem)

    def wait(p):
        _, _, _, rows, nrows, wec, wnc, sem = bufs[p]
        pltpu.make_async_copy(emb_hbm.at[pl.ds(0, _NBC)], rows, sem).wait()
        pltpu.make_async_copy(emb_hbm.at[pl.ds(0, _CL)], nrows, sem).wait()
        pltpu.make_async_copy(edge_hbm.at[pl.ds(0, _NBC)], wec, sem).wait()
        pltpu.make_async_copy(node_hbm.at[pl.ds(0, _CL)], wnc, sem).wait()

    def compute(p):
        _, _, flg, rows, nrows, wec, wnc, _ = bufs[p]

        def pair(l2, carry):
            wraw = wec[pl.ds(l2 * 16, 16)]
            fl = flg[pl.ds(l2 * 16, 16)]
            wpair = wraw + fl * (lastv - wraw)
            for h in range(2):
                l = l2 * 2 + h
                wvecs = [
                    jnp.broadcast_to(wpair[h * _G + g], (16,))
                    for g in range(_G)
                ]
                wn = plsc.load_gather(wnc, [jnp.full((16,), l, jnp.int32)])
                hi = jnp.full((16,), -65536, jnp.int32)
                for k2 in range(_D // 32):
                    sl = pl.ds(k2 * 16, 16)

                    def halves(r):
                        v = r[sl]
                        a = plsc.bitcast(v << 16, jnp.float32)
                        b = plsc.bitcast(v & hi, jnp.float32)
                        return a, b

                    a0, b0 = halves(rows.at[l * _G])
                    ma = wvecs[0] * a0
                    mb = wvecs[0] * b0
                    for g in range(1, _G):
                        ag, bg = halves(rows.at[l * _G + g])
                        ma = jnp.maximum(ma, wvecs[g] * ag)
                        mb = jnp.maximum(mb, wvecs[g] * bg)
                    na, nb2 = halves(nrows.at[l])
                    plsc.addupdate(accb.at[pl.ds(k2 * 32, 16)],
                                   ma + wn * (na - ma))
                    plsc.addupdate(accb.at[pl.ds(k2 * 32 + 16, 16)],
                                   mb + wn * (nb2 - mb))
            return carry

        lax.fori_loop(0, _CL // 2, pair, 0)

    for i in range(_BPW):
        b = wid * _BPW + i
        pltpu.sync_copy(nb_hbm.at[b], nbv)
        pltpu.sync_copy(we_hbm.at[b], wev)
        pltpu.sync_copy(x_hbm.at[b], xidx)

        for k in range(_D // 16):
            accb[pl.ds(k * 16, 16)] = jnp.zeros((16,), jnp.float32)
        for p in range(3):
            issue(jnp.int32(p), p)

        def outer(t, carry):
            j0 = 4 * t
            for p in range(4):
                wait(p)
                issue(jnp.minimum(j0 + p + 3, _NCHUNK - 1), (p + 3) % 4)
                compute(p)
            return carry

        lax.fori_loop(0, _NCHUNK // 4, outer, 0)
        for p in range(3):
            wait(p)
        pltpu.sync_copy(accb, out_hbm.at[b])


def _sc_aggregate(nbt, wet, x, emb, edge_flat, edge_last, node_w):
    mesh = plsc.VectorSubcoreMesh(core_axis_name="c", subcore_axis_name="s")
    return pl.kernel(
        _sc_body,
        out_type=jax.ShapeDtypeStruct((_B, _D), jnp.float32),
        mesh=mesh,
        scratch_types=[
            pltpu.VMEM((_G, _L), jnp.int32),
            pltpu.VMEM((_G, _L), jnp.int32),
            pltpu.VMEM((_L,), jnp.int32),
        ]
        + [pltpu.VMEM((_NBC,), jnp.int32)] * 4
        + [pltpu.VMEM((_NBC,), jnp.int32)] * 4
        + [pltpu.VMEM((_NBC,), jnp.float32)] * 4
        + [pltpu.VMEM((_NBC, _D // 2), jnp.int32)] * 4
        + [pltpu.VMEM((_CL, _D // 2), jnp.int32)] * 4
        + [pltpu.VMEM((_NBC,), jnp.float32)] * 4
        + [pltpu.VMEM((_CL,), jnp.float32)] * 4
        + [
            pltpu.VMEM((1, 1), jnp.float32),
            pltpu.VMEM((_D,), jnp.float32),
        ]
        + [pltpu.SemaphoreType.DMA] * 4,
        compiler_params=pltpu.CompilerParams(needs_layout_passes=False,
                                             use_tc_tiling_on_sc=False),
    )(nbt, wet, x, emb, edge_flat, edge_last, node_w)


def _fc_body(x_ref, w_ref, b_ref, o_ref):
    o_ref[...] = lax.dot_general(
        x_ref[...], w_ref[...],
        (((1,), (1,)), ((), ())),
        preferred_element_type=jnp.float32,
    ) + b_ref[...]


def kernel(x, nb_x, w_edge, emb, edge_w, node_w, fc_w, fc_b):
    x = x.astype(jnp.int32)
    nbt = jnp.transpose(nb_x.astype(jnp.int32), (0, 2, 1))
    wet = jnp.transpose(w_edge.astype(jnp.int32), (0, 2, 1))
    embp = jnp.transpose(
        emb.astype(jnp.bfloat16).reshape(8192, _D // 32, 2, 16),
        (0, 1, 3, 2)).reshape(8192, _D // 2, 2)
    embp = lax.bitcast_convert_type(embp, jnp.int32)
    edge_flat = lax.slice(edge_w, (0, 0), (_NE - 1, 1)).reshape(_NE - 1)
    edge_last = lax.slice(edge_w, (_NE - 1, 0), (_NE, 1))
    agg = _sc_aggregate(nbt, wet, x, embp, edge_flat, edge_last,
                        node_w.reshape(-1))
    y = pl.pallas_call(
        _fc_body,
        out_shape=jax.ShapeDtypeStruct((_B, _CAT), jnp.float32),
    )(agg, fc_w, fc_b.reshape(1, _CAT))
    return y


# speedup vs baseline: 1.0104x; 1.0104x over previous
import jax
import jax.numpy as jnp
from jax import lax
from jax.experimental import pallas as pl
from jax.experimental.pallas import tpu as pltpu
from jax.experimental.pallas import tpu_sc as plsc

_B = 64
_L = 512
_G = 8
_D = 128
_CAT = 20
_CL = 16
_NCHUNK = _L // _CL
_NBC = _CL * _G
_NC, _NS = 2, 16
_NW = _NC * _NS
_BPW = _B // _NW
_NE = (8192 - 1) * 8192 + 1


def _sc_body(nb_hbm, we_hbm, x_hbm, emb_hbm, edge_hbm, elast_hbm, node_hbm,
             out_hbm,
             nbv, wev, xidx,
             idxn0, idxn1, idxe0, idxe1, flg0, flg1,
             rows0, rows1, nrows0, nrows1, wec0, wec1, wnc0, wnc1,
             elv, accb, sem0, sem1):
    bufs = [
        (idxn0, idxe0, flg0, rows0, nrows0, wec0, wnc0, sem0),
        (idxn1, idxe1, flg1, rows1, nrows1, wec1, wnc1, sem1),
    ]
    wid = lax.axis_index("s") * _NC + lax.axis_index("c")
    lane = lax.broadcasted_iota(jnp.int32, (16,), 0)
    zero16 = jnp.zeros((16,), jnp.int32)
    pltpu.sync_copy(elast_hbm, elv)
    lastv = plsc.load_gather(elv, [zero16, zero16])

    def issue(j, p):
        idxn, idxe, flg, rows, nrows, wec, wnc, sem = bufs[p]
        for g in range(_G):
            sc_idx = lane * _G + g
            plsc.store_scatter(idxn, [sc_idx], nbv[g, pl.ds(j * _CL, _CL)])
            ev = wev[g, pl.ds(j * _CL, _CL)]
            plsc.store_scatter(idxe, [sc_idx], jnp.minimum(ev, _NE - 2))
            plsc.store_scatter(
                flg, [sc_idx],
                jnp.where(ev == _NE - 1,
                          jnp.full((16,), 1.0, jnp.float32),
                          jnp.zeros((16,), jnp.float32)))
        xsl = xidx.at[pl.ds(j * _CL, _CL)]
        pltpu.async_copy(emb_hbm.at[idxn], rows, sem)
        pltpu.async_copy(emb_hbm.at[xsl], nrows, sem)
        pltpu.async_copy(edge_hbm.at[idxe], wec, sem)
        pltpu.async_copy(node_hbm.at[xsl], wnc, sem)

    def wait(p):
        _, _, _, rows, nrows, wec, wnc, sem = bufs[p]
        pltpu.make_async_copy(emb_hbm.at[pl.ds(0, _NBC)], rows, sem).wait()
        pltpu.make_async_copy(emb_hbm.at[pl.ds(0, _CL)], nrows, sem).wait()
        pltpu.make_async_copy(edge_hbm.at[pl.ds(0, _NBC)], wec, sem).wait()
        pltpu.make_async_copy(node_hbm.at[pl.ds(0, _CL)], wnc, sem).wait()

    def compute(p):
        _, _, flg, rows, nrows, wec, wnc, _ = bufs[p]

        def pair(l2, carry):
            wraw = wec[pl.ds(l2 * 16, 16)]
            fl = flg[pl.ds(l2 * 16, 16)]
            wpair = wraw + fl * (lastv - wraw)
            for h in range(2):
                l = l2 * 2 + h
                wvecs = [
                    jnp.broadcast_to(wpair[h * _G + g], (16,))
                    for g in range(_G)
                ]
                wn = plsc.load_gather(wnc, [jnp.full((16,), l, jnp.int32)])
                hi = jnp.full((16,), -65536, jnp.int32)
                for k2 in range(_D // 32):
                    sl = pl.ds(k2 * 16, 16)

                    def halves(r):
                        v = r[sl]
                        a = plsc.bitcast(v << 16, jnp.float32)
                        b = plsc.bitcast(v & hi, jnp.float32)
                        return a, b

                    a0, b0 = halves(rows.at[l * _G])
                    ma = wvecs[0] * a0
                    mb = wvecs[0] * b0
                    for g in range(1, _G):
                        ag, bg = halves(rows.at[l * _G + g])
                        ma = jnp.maximum(ma, wvecs[g] * ag)
                        mb = jnp.maximum(mb, wvecs[g] * bg)
                    na, nb2 = halves(nrows.at[l])
                    plsc.addupdate(accb.at[pl.ds(k2 * 32, 16)],
                                   ma + wn * (na - ma))
                    plsc.addupdate(accb.at[pl.ds(k2 * 32 + 16, 16)],
                                   mb + wn * (nb2 - mb))
            return carry

        lax.fori_loop(0, _CL // 2, pair, 0)

    for i in range(_BPW):
        b = wid * _BPW + i
        pltpu.sync_copy(nb_hbm.at[b], nbv)
        pltpu.sync_copy(we_hbm.at[b], wev)
        pltpu.sync_copy(x_hbm.at[b], xidx)

        for k in range(_D // 16):
            accb[pl.ds(k * 16, 16)] = jnp.zeros((16,), jnp.float32)
        issue(jnp.int32(0), 0)

        def outer(t, carry):
            j0 = 2 * t
            wait(0)
            issue(jnp.minimum(j0 + 1, _NCHUNK - 1), 1)
            compute(0)
            wait(1)
            issue(jnp.minimum(j0 + 2, _NCHUNK - 1), 0)
            compute(1)
            return carry

        lax.fori_loop(0, _NCHUNK // 2, outer, 0)
        wait(0)
        pltpu.sync_copy(accb, out_hbm.at[b])


def _sc_aggregate(nbt, wet, x, emb, edge_flat, edge_last, node_w):
    mesh = plsc.VectorSubcoreMesh(core_axis_name="c", subcore_axis_name="s")
    return pl.kernel(
        _sc_body,
        out_type=jax.ShapeDtypeStruct((_B, _D), jnp.float32),
        mesh=mesh,
        scratch_types=[
            pltpu.VMEM((_G, _L), jnp.int32),
            pltpu.VMEM((_G, _L), jnp.int32),
            pltpu.VMEM((_L,), jnp.int32),
            pltpu.VMEM((_NBC,), jnp.int32),
            pltpu.VMEM((_NBC,), jnp.int32),
            pltpu.VMEM((_NBC,), jnp.int32),
            pltpu.VMEM((_NBC,), jnp.int32),
            pltpu.VMEM((_NBC,), jnp.float32),
            pltpu.VMEM((_NBC,), jnp.float32),
            pltpu.VMEM((_NBC, _D // 2), jnp.int32),
            pltpu.VMEM((_NBC, _D // 2), jnp.int32),
            pltpu.VMEM((_CL, _D // 2), jnp.int32),
            pltpu.VMEM((_CL, _D // 2), jnp.int32),
            pltpu.VMEM((_NBC,), jnp.float32),
            pltpu.VMEM((_NBC,), jnp.float32),
            pltpu.VMEM((_CL,), jnp.float32),
            pltpu.VMEM((_CL,), jnp.float32),
            pltpu.VMEM((1, 1), jnp.float32),
            pltpu.VMEM((_D,), jnp.float32),
            pltpu.SemaphoreType.DMA,
            pltpu.SemaphoreType.DMA,
        ],
        compiler_params=pltpu.CompilerParams(needs_layout_passes=False,
                                             use_tc_tiling_on_sc=False),
    )(nbt, wet, x, emb, edge_flat, edge_last, node_w)


def _fc_body(x_ref, w_ref, b_ref, o_ref):
    o_ref[...] = lax.dot_general(
        x_ref[...], w_ref[...],
        (((1,), (1,)), ((), ())),
        preferred_element_type=jnp.float32,
    ) + b_ref[...]


def kernel(x, nb_x, w_edge, emb, edge_w, node_w, fc_w, fc_b):
    x = x.astype(jnp.int32)
    nbt = jnp.transpose(nb_x.astype(jnp.int32), (0, 2, 1))
    wet = jnp.transpose(w_edge.astype(jnp.int32), (0, 2, 1))
    embp = jnp.transpose(
        emb.astype(jnp.bfloat16).reshape(8192, _D // 32, 2, 16),
        (0, 1, 3, 2)).reshape(8192, _D // 2, 2)
    embp = lax.bitcast_convert_type(embp, jnp.int32)
    edge_flat = lax.slice(edge_w, (0, 0), (_NE - 1, 1)).reshape(_NE - 1)
    edge_last = lax.slice(edge_w, (_NE - 1, 0), (_NE, 1))
    agg = _sc_aggregate(nbt, wet, x, embp, edge_flat, edge_last,
                        node_w.reshape(-1))
    y = pl.pallas_call(
        _fc_body,
        out_shape=jax.ShapeDtypeStruct((_B, _CAT), jnp.float32),
    )(agg, fc_w, fc_b.reshape(1, _CAT))
    return y
